# baseline (speedup 1.0000x reference)
import functools

import jax
import jax.numpy as jnp
import numpy as np
from jax import lax
from jax.experimental import pallas as pl
from jax.experimental.pallas import tpu as pltpu
from jax.experimental.pallas import tpu_sc as plsc

NC = 2
NS = 16
LANES = 16
NW = NC * NS

R = 8192
C = 4096
ROWS_W = R // NW
RCHUNK = 8
NCHUNK = ROWS_W // RCHUNK
BRCHUNK = 4
BNCHUNK = ROWS_W // BRCHUNK
BCHUNK = BRCHUNK * C
CHUNK = RCHUNK * C
NWIN = BCHUNK // 128

MAGICF = np.float32(12582912.0)
MAGICI = np.int32(0x4B400000)
SIGNBIT = np.int32(np.uint32(0x80000000).view(np.int32))

_mesh = plsc.VectorSubcoreMesh(core_axis_name="c", subcore_axis_name="s")
_params = pltpu.CompilerParams(needs_layout_passes=False)


def _wid():
    return lax.axis_index("s") * NC + lax.axis_index("c")


def _rows_in(x_hbm, rr, buf, sem, nrows=RCHUNK):
    for s in range(nrows):
        pltpu.async_copy(x_hbm.at[rr + s, :], buf.at[pl.ds(s * C, C)], sem)


def _rows_in_wait(x_hbm, rr, buf, sem, nrows=RCHUNK):
    for s in range(nrows):
        pltpu.make_async_copy(x_hbm.at[rr + s, :], buf.at[pl.ds(s * C, C)],
                              sem).wait()


def _rows_out(out_hbm, rr, buf, sem, nrows=RCHUNK):
    for s in range(nrows):
        pltpu.async_copy(buf.at[pl.ds(s * C, C)], out_hbm.at[rr + s, :], sem)


def _rows_out_wait(out_hbm, rr, buf, sem, nrows=RCHUNK):
    for s in range(nrows):
        pltpu.make_async_copy(buf.at[pl.ds(s * C, C)], out_hbm.at[rr + s, :],
                              sem).wait()


@functools.partial(
    pl.kernel,
    out_type=(
        jax.ShapeDtypeStruct((NW * LANES,), jnp.float32),
        jax.ShapeDtypeStruct((NW * LANES,), jnp.float32),
    ),
    mesh=_mesh,
    compiler_params=_params,
    scratch_types=[
        pltpu.VMEM((CHUNK,), jnp.float32),
        pltpu.VMEM((CHUNK,), jnp.float32),
        pltpu.VMEM((LANES,), jnp.float32),
        pltpu.VMEM((LANES,), jnp.float32),
        pltpu.SemaphoreType.DMA,
        pltpu.SemaphoreType.DMA,
    ],
)
def _minmax_kernel(x_hbm, min_hbm, max_hbm, in0, in1, mn_buf, mx_buf, si0, si1):
    wid = _wid()
    r0 = wid * ROWS_W
    bufs = ((in0, si0), (in1, si1))

    _rows_in(x_hbm, r0, in0, si0)
    _rows_in(x_hbm, r0 + RCHUNK, in1, si1)

    def pair(p, carry):
        for b, (inb, si) in enumerate(bufs):
            ci = p * 2 + b
            rr = r0 + ci * RCHUNK
            _rows_in_wait(x_hbm, rr, inb, si)

            def vred(i, acc):
                off = i * 128
                new = []
                for k in range(8):
                    v = inb[pl.ds(off + k * LANES, LANES)]
                    new.append(jnp.minimum(acc[k], v))
                for k in range(8):
                    v = inb[pl.ds(off + k * LANES, LANES)]
                    new.append(jnp.maximum(acc[8 + k], v))
                return tuple(new)

            carry = lax.fori_loop(0, CHUNK // 128, vred, carry)

            @pl.when(ci + 2 < NCHUNK)
            def _():
                _rows_in(x_hbm, rr + 2 * RCHUNK, inb, si)
        return carry

    init = tuple(
        jnp.full((LANES,), jnp.inf if k < 8 else -jnp.inf, jnp.float32)
        for k in range(16)
    )
    acc = lax.fori_loop(0, NCHUNK // 2, pair, init)
    mn = acc[0]
    mx = acc[8]
    for k in range(1, 8):
        mn = jnp.minimum(mn, acc[k])
        mx = jnp.maximum(mx, acc[8 + k])
    mn_buf[...] = mn
    mx_buf[...] = mx
    pltpu.sync_copy(mn_buf, min_hbm.at[pl.ds(wid * LANES, LANES)])
    pltpu.sync_copy(mx_buf, max_hbm.at[pl.ds(wid * LANES, LANES)])


def _ce(a, b):
    return jnp.maximum(a, b), jnp.minimum(a, b)


def _windows(inb, outb, sv, iv):
    idx8 = lax.iota(jnp.int32, LANES) * 8

    @plsc.parallel_loop(0, NWIN, 1, unroll=4)
    def window(w):
        base = w * 128
        idxs = [idx8 + (base + j) for j in range(8)]
        xs = [plsc.load_gather(inb, [idxs[j]]) for j in range(8)]
        ms = []
        for j in range(8):
            t = xs[j] * iv
            u = t + MAGICF
            q0 = plsc.bitcast(u, jnp.int32) - MAGICI
            ms.append(jnp.abs(jnp.minimum(jnp.maximum(q0, -128), 127)))
        a0, a1 = _ce(ms[0], ms[1])
        a2, a3 = _ce(ms[2], ms[3])
        a0, a2 = _ce(a0, a2)
        a1, a3 = _ce(a1, a3)
        a1, a2 = _ce(a1, a2)
        b0, b1 = _ce(ms[4], ms[5])
        b2, b3 = _ce(ms[6], ms[7])
        b0, b2 = _ce(b0, b2)
        b1, b3 = _ce(b1, b3)
        b1, b2 = _ce(b1, b2)
        thr = jnp.maximum(
            jnp.maximum(b3, jnp.minimum(a0, b2)),
            jnp.maximum(jnp.minimum(a1, b1),
                        jnp.maximum(jnp.minimum(a2, b0), a3)))
        for j in range(8):
            hi = ms[j] & -16
            k2 = jnp.where(ms[j] >= thr, ms[j], hi)
            fs = k2.astype(jnp.float32) * sv
            ob = plsc.bitcast(
                plsc.bitcast(fs, jnp.int32)
                | (plsc.bitcast(xs[j], jnp.int32) & SIGNBIT),
                jnp.float32)
            plsc.store_scatter(outb, [idxs[j]], ob)


@functools.partial(
    pl.kernel,
    out_type=jax.ShapeDtypeStruct((R, C), jnp.float32),
    mesh=_mesh,
    compiler_params=_params,
    scratch_types=[
        pltpu.VMEM((BCHUNK,), jnp.float32),
        pltpu.VMEM((BCHUNK,), jnp.float32),
        pltpu.VMEM((BCHUNK,), jnp.float32),
        pltpu.VMEM((BCHUNK,), jnp.float32),
        pltpu.VMEM((NW * LANES,), jnp.float32),
        pltpu.VMEM((NW * LANES,), jnp.float32),
        pltpu.SemaphoreType.DMA,
        pltpu.SemaphoreType.DMA,
        pltpu.SemaphoreType.DMA,
        pltpu.SemaphoreType.DMA,
    ],
)
def _quant_kernel(x_hbm, min_hbm, max_hbm, out_hbm,
                  in0, in1, out0, out1, mnb, mxb, si0, si1, so0, so1):
    wid = _wid()
    r0 = wid * ROWS_W
    bufs = ((in0, out0, si0, so0), (in1, out1, si1, so1))

    _rows_in(x_hbm, r0, in0, si0, BRCHUNK)
    _rows_in(x_hbm, r0 + BRCHUNK, in1, si1, BRCHUNK)

    pltpu.sync_copy(min_hbm, mnb)
    pltpu.sync_copy(max_hbm, mxb)
    mnv = mnb[pl.ds(0, LANES)]
    mxv = mxb[pl.ds(0, LANES)]
    for w in range(1, NW):
        mnv = jnp.minimum(mnv, mnb[pl.ds(w * LANES, LANES)])
        mxv = jnp.maximum(mxv, mxb[pl.ds(w * LANES, LANES)])
    rmin = jnp.full((LANES,), jnp.min(mnv), jnp.float32)
    rmax = jnp.full((LANES,), jnp.max(mxv), jnp.float32)
    rmin = jnp.minimum(rmin, 0.0)
    rmax = jnp.maximum(rmax, 0.0)
    sv = jnp.maximum((rmax - rmin) / 255.0, 1e-8)
    iv = jnp.float32(1.0) / sv

    def pair(p, _):
        for b, (inb, outb, si, so) in enumerate(bufs):
            ci = p * 2 + b
            rr = r0 + ci * BRCHUNK
            _rows_in_wait(x_hbm, rr, inb, si, BRCHUNK)

            @pl.when(ci >= 2)
            def _():
                _rows_out_wait(out_hbm, r0, outb, so, BRCHUNK)

            _windows(inb, outb, sv, iv)
            _rows_out(out_hbm, rr, outb, so, BRCHUNK)

            @pl.when(ci + 2 < BNCHUNK)
            def _():
                _rows_in(x_hbm, rr + 2 * BRCHUNK, inb, si, BRCHUNK)
        return 0

    lax.fori_loop(0, BNCHUNK // 2, pair, 0)
    _rows_out_wait(out_hbm, r0, out0, so0, BRCHUNK)
    _rows_out_wait(out_hbm, r0, out1, so1, BRCHUNK)


def kernel(x):
    x2 = x.reshape(R, C)
    mn, mx = _minmax_kernel(x2)
    out = _quant_kernel(x2, mn, mx)
    return out.reshape(x.shape)


# speedup vs baseline: 1.5053x; 1.5053x over previous
import functools

import jax
import jax.numpy as jnp
import numpy as np
from jax import lax
from jax.experimental import pallas as pl
from jax.experimental.pallas import tpu as pltpu
from jax.experimental.pallas import tpu_sc as plsc

NC = 2
NS = 16
LANES = 16
NW = NC * NS

R = 8192
C = 4096
ROWS_W = R // NW
RCHUNK = 8
NCHUNK = ROWS_W // RCHUNK
BRCHUNK = 4
BNCHUNK = ROWS_W // BRCHUNK
BCHUNK = BRCHUNK * C
CHUNK = RCHUNK * C
NWIN = BCHUNK // 128

MAGICF = np.float32(12582912.0)
MAGICI = np.int32(0x4B400000)
SIGNBIT = np.int32(np.uint32(0x80000000).view(np.int32))

_mesh = plsc.VectorSubcoreMesh(core_axis_name="c", subcore_axis_name="s")
_params = pltpu.CompilerParams(needs_layout_passes=False)


def _wid():
    return lax.axis_index("s") * NC + lax.axis_index("c")


def _rows_in(x_hbm, rr, buf, sem, nrows=RCHUNK):
    for s in range(nrows):
        pltpu.async_copy(x_hbm.at[rr + s, :], buf.at[pl.ds(s * C, C)], sem)


def _rows_in_wait(x_hbm, rr, buf, sem, nrows=RCHUNK):
    for s in range(nrows):
        pltpu.make_async_copy(x_hbm.at[rr + s, :], buf.at[pl.ds(s * C, C)],
                              sem).wait()


def _rows_out(out_hbm, rr, buf, sem, nrows=RCHUNK):
    for s in range(nrows):
        pltpu.async_copy(buf.at[pl.ds(s * C, C)], out_hbm.at[rr + s, :], sem)


def _rows_out_wait(out_hbm, rr, buf, sem, nrows=RCHUNK):
    for s in range(nrows):
        pltpu.make_async_copy(buf.at[pl.ds(s * C, C)], out_hbm.at[rr + s, :],
                              sem).wait()


@functools.partial(
    pl.kernel,
    out_type=(
        jax.ShapeDtypeStruct((NW * LANES,), jnp.float32),
        jax.ShapeDtypeStruct((NW * LANES,), jnp.float32),
    ),
    mesh=_mesh,
    compiler_params=_params,
    scratch_types=[
        pltpu.VMEM((CHUNK,), jnp.float32),
        pltpu.VMEM((CHUNK,), jnp.float32),
        pltpu.VMEM((LANES,), jnp.float32),
        pltpu.VMEM((LANES,), jnp.float32),
        pltpu.SemaphoreType.DMA,
        pltpu.SemaphoreType.DMA,
    ],
)
def _minmax_kernel(x_hbm, min_hbm, max_hbm, in0, in1, mn_buf, mx_buf, si0, si1):
    wid = _wid()
    r0 = wid * ROWS_W
    bufs = ((in0, si0), (in1, si1))

    _rows_in(x_hbm, r0, in0, si0)
    _rows_in(x_hbm, r0 + RCHUNK, in1, si1)

    def pair(p, carry):
        for b, (inb, si) in enumerate(bufs):
            ci = p * 2 + b
            rr = r0 + ci * RCHUNK
            _rows_in_wait(x_hbm, rr, inb, si)

            def vred(i, acc):
                off = i * 128
                new = []
                for k in range(8):
                    v = inb[pl.ds(off + k * LANES, LANES)]
                    new.append(jnp.minimum(acc[k], v))
                for k in range(8):
                    v = inb[pl.ds(off + k * LANES, LANES)]
                    new.append(jnp.maximum(acc[8 + k], v))
                return tuple(new)

            carry = lax.fori_loop(0, CHUNK // 128, vred, carry)

            @pl.when(ci + 2 < NCHUNK)
            def _():
                _rows_in(x_hbm, rr + 2 * RCHUNK, inb, si)
        return carry

    init = tuple(
        jnp.full((LANES,), jnp.inf if k < 8 else -jnp.inf, jnp.float32)
        for k in range(16)
    )
    acc = lax.fori_loop(0, NCHUNK // 2, pair, init)
    mn = acc[0]
    mx = acc[8]
    for k in range(1, 8):
        mn = jnp.minimum(mn, acc[k])
        mx = jnp.maximum(mx, acc[8 + k])
    mn_buf[...] = mn
    mx_buf[...] = mx
    pltpu.sync_copy(mn_buf, min_hbm.at[pl.ds(wid * LANES, LANES)])
    pltpu.sync_copy(mx_buf, max_hbm.at[pl.ds(wid * LANES, LANES)])


def _ce(a, b):
    return jnp.maximum(a, b), jnp.minimum(a, b)


def _windows(inb, outb, sv, iv):
    idx8 = lax.iota(jnp.int32, LANES) * 8

    @plsc.parallel_loop(0, NWIN, 1, unroll=2)
    def window(w):
        base = w * 128
        idxs = [idx8 + (base + j) for j in range(8)]
        xs = [plsc.load_gather(inb, [idxs[j]]) for j in range(8)]
        ms = []
        for j in range(8):
            t = xs[j] * iv
            u = t + MAGICF
            q0 = plsc.bitcast(u, jnp.int32) - MAGICI
            ms.append(jnp.abs(jnp.minimum(jnp.maximum(q0, -128), 127)))
        a0, a1 = _ce(ms[0], ms[1])
        a2, a3 = _ce(ms[2], ms[3])
        a0, a2 = _ce(a0, a2)
        a1, a3 = _ce(a1, a3)
        a1, a2 = _ce(a1, a2)
        b0, b1 = _ce(ms[4], ms[5])
        b2, b3 = _ce(ms[6], ms[7])
        b0, b2 = _ce(b0, b2)
        b1, b3 = _ce(b1, b3)
        b1, b2 = _ce(b1, b2)
        thr = jnp.maximum(
            jnp.maximum(b3, jnp.minimum(a0, b2)),
            jnp.maximum(jnp.minimum(a1, b1),
                        jnp.maximum(jnp.minimum(a2, b0), a3)))
        for j in range(8):
            hi = ms[j] & -16
            k2 = jnp.where(ms[j] >= thr, ms[j], hi)
            fs = k2.astype(jnp.float32) * sv
            ob = plsc.bitcast(
                plsc.bitcast(fs, jnp.int32)
                | (plsc.bitcast(xs[j], jnp.int32) & SIGNBIT),
                jnp.float32)
            plsc.store_scatter(outb, [idxs[j]], ob)


@functools.partial(
    pl.kernel,
    out_type=jax.ShapeDtypeStruct((R, C), jnp.float32),
    mesh=_mesh,
    compiler_params=_params,
    scratch_types=[
        pltpu.VMEM((BCHUNK,), jnp.float32),
        pltpu.VMEM((BCHUNK,), jnp.float32),
        pltpu.VMEM((BCHUNK,), jnp.float32),
        pltpu.VMEM((BCHUNK,), jnp.float32),
        pltpu.VMEM((NW * LANES,), jnp.float32),
        pltpu.VMEM((NW * LANES,), jnp.float32),
        pltpu.SemaphoreType.DMA,
        pltpu.SemaphoreType.DMA,
        pltpu.SemaphoreType.DMA,
        pltpu.SemaphoreType.DMA,
    ],
)
def _quant_kernel(x_hbm, min_hbm, max_hbm, out_hbm,
                  in0, in1, out0, out1, mnb, mxb, si0, si1, so0, so1):
    wid = _wid()
    r0 = wid * ROWS_W
    bufs = ((in0, out0, si0, so0), (in1, out1, si1, so1))

    _rows_in(x_hbm, r0, in0, si0, BRCHUNK)
    _rows_in(x_hbm, r0 + BRCHUNK, in1, si1, BRCHUNK)

    pltpu.sync_copy(min_hbm, mnb)
    pltpu.sync_copy(max_hbm, mxb)
    mnv = mnb[pl.ds(0, LANES)]
    mxv = mxb[pl.ds(0, LANES)]
    for w in range(1, NW):
        mnv = jnp.minimum(mnv, mnb[pl.ds(w * LANES, LANES)])
        mxv = jnp.maximum(mxv, mxb[pl.ds(w * LANES, LANES)])
    rmin = jnp.full((LANES,), jnp.min(mnv), jnp.float32)
    rmax = jnp.full((LANES,), jnp.max(mxv), jnp.float32)
    rmin = jnp.minimum(rmin, 0.0)
    rmax = jnp.maximum(rmax, 0.0)
    sv = jnp.maximum((rmax - rmin) / 255.0, 1e-8)
    iv = jnp.float32(1.0) / sv

    def pair(p, _):
        for b, (inb, outb, si, so) in enumerate(bufs):
            ci = p * 2 + b
            rr = r0 + ci * BRCHUNK
            _rows_in_wait(x_hbm, rr, inb, si, BRCHUNK)

            @pl.when(ci >= 2)
            def _():
                _rows_out_wait(out_hbm, r0, outb, so, BRCHUNK)

            _windows(inb, outb, sv, iv)
            _rows_out(out_hbm, rr, outb, so, BRCHUNK)

            @pl.when(ci + 2 < BNCHUNK)
            def _():
                _rows_in(x_hbm, rr + 2 * BRCHUNK, inb, si, BRCHUNK)
        return 0

    lax.fori_loop(0, BNCHUNK // 2, pair, 0)
    _rows_out_wait(out_hbm, r0, out0, so0, BRCHUNK)
    _rows_out_wait(out_hbm, r0, out1, so1, BRCHUNK)


def kernel(x):
    x2 = x.reshape(R, C)
    mn, mx = _minmax_kernel(x2)
    out = _quant_kernel(x2, mn, mx)
    return out.reshape(x.shape)
